# baseline (speedup 1.0000x reference)
import functools

import jax
import jax.numpy as jnp
from jax import lax
from jax.experimental import pallas as pl
from jax.experimental.pallas import tpu as pltpu
from jax.experimental.pallas import tpu_sc as plsc

_WAVE = 16


@functools.cache
def _make_gather(V, D, B):
    info = plsc.get_sparse_core_info()
    NC, NS = info.num_cores, info.num_subcores
    NW = NC * NS
    n_ranges = NW // 2
    assert B % (_WAVE * n_ranges) == 0, (V, D, B, NW)
    b_per_r = B // n_ranges
    n_waves = b_per_r // _WAVE
    half = D // 2
    mesh = plsc.VectorSubcoreMesh(core_axis_name="c", subcore_axis_name="s")

    @functools.partial(
        pl.kernel,
        mesh=mesh,
        compiler_params=pltpu.CompilerParams(
            use_tc_tiling_on_sc=True, needs_layout_passes=False
        ),
        out_type=jax.ShapeDtypeStruct((D, B), jnp.float32),
        scratch_types=[
            pltpu.VMEM((b_per_r,), jnp.int32),
            pltpu.VMEM((half, _WAVE * 128), jnp.float32),
            pltpu.VMEM((half, b_per_r), jnp.float32),
            pltpu.SemaphoreType.DMA,
        ],
    )
    def gather_kernel(table_hbm, idx_hbm, out_hbm, idx_v, ring_v, cols_v, sem):
        wid = lax.axis_index("s") * NC + lax.axis_index("c")
        rng = wid % n_ranges
        rlo = pl.multiple_of((wid // n_ranges) * half, 8)
        base = rng * b_per_r
        pltpu.sync_copy(idx_hbm.at[pl.ds(base, b_per_r)], idx_v)

        def wave(g):
            v = idx_v[pl.ds(g * _WAVE, _WAVE)]
            for j in range(_WAVE):
                tcol = pl.multiple_of((v[j] >> 7) << 7, 128)
                pltpu.async_copy(
                    table_hbm.at[pl.ds(rlo, half), pl.ds(tcol, 128)],
                    ring_v.at[:, pl.ds(j * 128, 128)],
                    sem,
                )
            pltpu.make_async_copy(
                table_hbm.at[pl.ds(0, half), pl.ds(0, _WAVE * 128)], ring_v, sem
            ).wait()
            lane = v & 127
            src_col = lax.iota(jnp.int32, _WAVE) * 128 + lane
            dst_col = g * _WAVE + lax.iota(jnp.int32, _WAVE)
            for j in range(half):
                row = jnp.full((_WAVE,), j, jnp.int32)
                vals = plsc.load_gather(ring_v, [row, src_col])
                plsc.store_scatter(cols_v, [row, dst_col], vals)

        pl.loop(0, n_waves)(wave)
        pltpu.sync_copy(cols_v, out_hbm.at[pl.ds(rlo, half), pl.ds(base, b_per_r)])

    return gather_kernel


def kernel(graph, feat, nodes, emb):
    V, D = emb.shape
    (B,) = nodes.shape
    out_t = _make_gather(V, D, B)(emb.T, nodes)
    return out_t.T


# speedup vs baseline: 1.1716x; 1.1716x over previous
import functools

import jax
import jax.numpy as jnp
from jax import lax
from jax.experimental import pallas as pl
from jax.experimental.pallas import tpu as pltpu
from jax.experimental.pallas import tpu_sc as plsc

_LANES = 16
_WAVE = 16


@functools.cache
def _make_gather(V, D, B):
    info = plsc.get_sparse_core_info()
    NC, NS = info.num_cores, info.num_subcores
    NW = NC * NS
    assert B % (_LANES * NW) == 0, (V, D, B, NW)
    b_per_w = B // NW
    n_waves = b_per_w // _WAVE
    mesh = plsc.VectorSubcoreMesh(core_axis_name="c", subcore_axis_name="s")

    @functools.partial(
        pl.kernel,
        mesh=mesh,
        compiler_params=pltpu.CompilerParams(
            use_tc_tiling_on_sc=True, needs_layout_passes=False
        ),
        out_type=jax.ShapeDtypeStruct((D, B), jnp.float32),
        scratch_types=[
            pltpu.VMEM((b_per_w,), jnp.int32),
            pltpu.VMEM((D, _WAVE * 128), jnp.float32),
            pltpu.VMEM((D, b_per_w), jnp.float32),
            pltpu.SemaphoreType.DMA,
        ],
    )
    def gather_kernel(table_hbm, idx_hbm, out_hbm, idx_v, ring_v, cols_v, sem):
        wid = lax.axis_index("s") * NC + lax.axis_index("c")
        base = wid * b_per_w
        pltpu.sync_copy(idx_hbm.at[pl.ds(base, b_per_w)], idx_v)

        def wave(g):
            v = idx_v[pl.ds(g * _WAVE, _WAVE)]
            for j in range(_WAVE):
                tcol = pl.multiple_of((v[j] >> 7) << 7, 128)
                pltpu.async_copy(
                    table_hbm.at[:, pl.ds(tcol, 128)],
                    ring_v.at[:, pl.ds(j * 128, 128)],
                    sem,
                )
            pltpu.make_async_copy(
                table_hbm.at[:, pl.ds(0, _WAVE * 128)], ring_v, sem
            ).wait()
            lane = v & 127
            slot_base = lax.iota(jnp.int32, _LANES) * 128
            src_col = slot_base + lane
            dst_col = g * _WAVE + lax.iota(jnp.int32, _LANES)
            for j in range(D):
                row = jnp.full((_LANES,), j, jnp.int32)
                vals = plsc.load_gather(ring_v, [row, src_col])
                plsc.store_scatter(cols_v, [row, dst_col], vals)

        pl.loop(0, n_waves)(wave)
        pltpu.sync_copy(cols_v, out_hbm.at[:, pl.ds(base, b_per_w)])

    return gather_kernel


def kernel(graph, feat, nodes, emb):
    V, D = emb.shape
    (B,) = nodes.shape
    out_t = _make_gather(V, D, B)(emb.T, nodes)
    return out_t.T
